# baseline (speedup 1.0000x reference)
import dataclasses
import functools

import jax
import jax.numpy as jnp
from jax import lax
from jax.experimental import pallas as pl
from jax.experimental.pallas import tpu as pltpu
from jax.experimental.pallas import tpu_sc as plsc

G = 64
C = 2
D = 64
H = 64
W = 64
B = 131072

NC = 2
NS = 16
LANES = 16
NW = NC * NS
BPW = B // NW
TB = 256
NTILES = BPW // TB
NCHUNK = TB // LANES


def _floor_vec(v):
    t = v.astype(jnp.int32).astype(jnp.float32)
    return t - jnp.where(t > v, 1.0, 0.0).astype(jnp.float32)


def _axis_weights(coord, n):
    i = (coord + 1.0) * (0.5 * (n - 1))
    i = jnp.minimum(jnp.maximum(i, -2.0), float(n + 2))
    f = _floor_vec(i)
    frac = i - f
    one = jnp.float32(1.0)
    zero = jnp.float32(0.0)
    in0 = jnp.where((f >= 0.0) & (f <= n - 1.0), one, zero)
    in1 = jnp.where((f >= -1.0) & (f <= n - 2.0), one, zero)
    w0 = (one - frac) * in0
    w1 = frac * in1
    neg = f < 0.0
    s0 = jnp.where(neg, w1, w0)
    s1 = jnp.where(neg, zero, w1)
    cell = jnp.minimum(jnp.maximum(f, 0.0), n - 1.0).astype(jnp.int32)
    return cell, s0, s1


def _sc_interp(table, tpx, tpy, tpz):
    mesh = plsc.VectorSubcoreMesh(core_axis_name="c", subcore_axis_name="s")
    cp = pltpu.CompilerParams()
    if "needs_layout_passes" in pltpu.CompilerParams.__dataclass_fields__:
        cp = dataclasses.replace(cp, needs_layout_passes=False)
    if "use_tc_tiling_on_sc" in pltpu.CompilerParams.__dataclass_fields__:
        cp = dataclasses.replace(cp, use_tc_tiling_on_sc=False)

    @functools.partial(
        pl.kernel,
        out_type=jax.ShapeDtypeStruct((B, G * C), jnp.float32),
        mesh=mesh,
        compiler_params=cp,
        scratch_types=[
            pltpu.VMEM((TB,), jnp.float32),
            pltpu.VMEM((TB,), jnp.float32),
            pltpu.VMEM((TB,), jnp.float32),
            pltpu.VMEM((6, TB), jnp.float32),
            pltpu.VMEM((2, TB // 2), jnp.int32),
            pltpu.VMEM((TB, 16), jnp.float32),
            pltpu.VMEM((TB, G * C), jnp.float32),
            pltpu.SemaphoreType.DMA,
            pltpu.SemaphoreType.DMA,
        ],
    )
    def k(tab_hbm, tpx_hbm, tpy_hbm, tpz_hbm, out_hbm,
          cx, cy, cz, wbuf, idxbuf, gbuf, obuf, csem, gsem):
        wid = lax.axis_index("s") * NC + lax.axis_index("c")
        base = wid * BPW
        iota = lax.iota(jnp.int32, LANES)

        @pl.loop(0, NTILES)
        def _tile(bt):
            boff = base + bt * TB

            @pl.loop(0, G)
            def _grid(g):
                c1 = pltpu.async_copy(tpx_hbm.at[g, pl.ds(boff, TB)], cx, csem)
                c2 = pltpu.async_copy(tpy_hbm.at[g, pl.ds(boff, TB)], cy, csem)
                c3 = pltpu.async_copy(tpz_hbm.at[g, pl.ds(boff, TB)], cz, csem)
                c1.wait()
                c2.wait()
                c3.wait()

                goff = g * (D * H * W)

                @pl.loop(0, NCHUNK)
                def _idx(ci):
                    sl = pl.ds(ci * LANES, LANES)
                    xc, sx0, sx1 = _axis_weights(cx[sl], W)
                    yc, sy0, sy1 = _axis_weights(cy[sl], H)
                    zc, sz0, sz1 = _axis_weights(cz[sl], D)
                    idx = goff + (
                        lax.shift_left(zc, 12)
                        + lax.shift_left(yc, 6)
                        + xc
                    )
                    idxbuf[ci // 8, pl.ds((ci % 8) * LANES, LANES)] = idx
                    wbuf[0, sl] = sx0
                    wbuf[1, sl] = sx1
                    wbuf[2, sl] = sy0
                    wbuf[3, sl] = sy1
                    wbuf[4, sl] = sz0
                    wbuf[5, sl] = sz1

                g1 = pltpu.async_copy(
                    tab_hbm.at[idxbuf.at[0]], gbuf.at[pl.ds(0, TB // 2)], gsem)
                g2 = pltpu.async_copy(
                    tab_hbm.at[idxbuf.at[1]], gbuf.at[pl.ds(TB // 2, TB // 2)], gsem)
                g1.wait()
                g2.wait()

                @pl.loop(0, NCHUNK)
                def _reduce(ci):
                    sl = pl.ds(ci * LANES, LANES)
                    sx0 = wbuf[0, sl]
                    sx1 = wbuf[1, sl]
                    sy0 = wbuf[2, sl]
                    sy1 = wbuf[3, sl]
                    sz0 = wbuf[4, sl]
                    sz1 = wbuf[5, sl]
                    w00 = sz0 * sy0
                    w01 = sz0 * sy1
                    w10 = sz1 * sy0
                    w11 = sz1 * sy1
                    wc = (w00 * sx0, w00 * sx1, w01 * sx0, w01 * sx1,
                          w10 * sx0, w10 * sx1, w11 * sx0, w11 * sx1)
                    rows = iota + ci * LANES
                    acc0 = jnp.zeros((LANES,), jnp.float32)
                    acc1 = jnp.zeros((LANES,), jnp.float32)
                    for corner in range(8):
                        col0 = jnp.full((LANES,), 2 * corner, jnp.int32)
                        col1 = jnp.full((LANES,), 2 * corner + 1, jnp.int32)
                        v0 = plsc.load_gather(gbuf, [rows, col0])
                        v1 = plsc.load_gather(gbuf, [rows, col1])
                        acc0 = acc0 + wc[corner] * v0
                        acc1 = acc1 + wc[corner] * v1
                    oc0 = jnp.full((LANES,), 2 * g, jnp.int32)
                    oc1 = jnp.full((LANES,), 2 * g + 1, jnp.int32)
                    plsc.store_scatter(obuf, [rows, oc0], acc0)
                    plsc.store_scatter(obuf, [rows, oc1], acc1)

            pltpu.sync_copy(obuf, out_hbm.at[pl.ds(boff, TB)])

    return k(table, tpx, tpy, tpz)


@jax.jit
def kernel(x, transformation_matrices, feature_grids):
    tm = transformation_matrices
    pts = jnp.concatenate(
        [x, jnp.ones((x.shape[0], 1), dtype=x.dtype)], axis=1)
    tp = jnp.einsum("gij,bj->gib", tm[:, 0:3, :], pts)
    tpx = tp[:, 0, :]
    tpy = tp[:, 1, :]
    tpz = tp[:, 2, :]

    fgt = jnp.transpose(feature_grids, (0, 2, 3, 4, 1))

    def shift(a, axis):
        n = a.shape[axis]
        lead = lax.slice_in_dim(a, 1, n, axis=axis)
        edge = lax.slice_in_dim(a, n - 1, n, axis=axis)
        return jnp.concatenate([lead, edge], axis=axis)

    slots = []
    for zo in range(2):
        az = fgt if zo == 0 else shift(fgt, 1)
        for yo in range(2):
            ay = az if yo == 0 else shift(az, 2)
            for xo in range(2):
                ax = ay if xo == 0 else shift(ay, 3)
                slots.append(ax)
    table = jnp.stack(slots, axis=4)
    table = table.reshape(G * D * H * W, 16)

    return _sc_interp(table, tpx, tpy, tpz)
